# baseline (speedup 1.0000x reference)
import jax
import jax.numpy as jnp
from jax import lax
from jax.experimental import pallas as pl

_B = 4
_N = 2048
_NS1, _NS1P = 102, 104
_NS2, _NS2P = 5, 8
_K = 32
_R1, _R2, _R3 = 0.3, 1.0, 2.0
_NBD, _NB2, _NB1 = 5, 20, 20
_PAD = 1.0e4
_BIG = 1.0e30
_NEG = -1.0e30


def _transpose(a):
    c = a.shape[1]
    eye = (lax.broadcasted_iota(jnp.int32, (c, c), 0)
           == lax.broadcasted_iota(jnp.int32, (c, c), 1)).astype(jnp.float32)
    return lax.dot_general(eye, a, (((1,), (1,)), ((), ())),
                           preferred_element_type=jnp.float32,
                           precision=lax.Precision.HIGHEST)


def _fps(cx, cy, cz, nsamp, nslots, lane_valid):
    bb, ll = cx.shape
    lane = lax.broadcasted_iota(jnp.int32, (bb, ll), 1)
    slot = lax.broadcasted_iota(jnp.int32, (3 * bb, nslots), 1)
    p0x, p0y, p0z = cx[:, 0:1], cy[:, 0:1], cz[:, 0:1]
    d0 = (cx - p0x) ** 2 + (cy - p0y) ** 2 + (cz - p0z) ** 2
    if lane_valid is not None:
        d0 = jnp.where(lane_valid, d0, _NEG)
    sacc0 = jnp.where(slot == 0,
                      jnp.concatenate([p0x, p0y, p0z], axis=0),
                      jnp.float32(_PAD))

    def body(i, carry):
        d, sacc = carry
        m = jnp.max(d, axis=1, keepdims=True)
        idx = jnp.min(jnp.where(d == m, lane, ll), axis=1, keepdims=True)
        oh = lane == idx
        px = jnp.sum(jnp.where(oh, cx, 0.0), axis=1, keepdims=True)
        py = jnp.sum(jnp.where(oh, cy, 0.0), axis=1, keepdims=True)
        pz = jnp.sum(jnp.where(oh, cz, 0.0), axis=1, keepdims=True)
        nd = (cx - px) ** 2 + (cy - py) ** 2 + (cz - pz) ** 2
        d = jnp.minimum(d, nd)
        sacc = jnp.where(slot == i,
                         jnp.concatenate([px, py, pz], axis=0), sacc)
        return d, sacc

    _, sacc = lax.fori_loop(1, nsamp, body, (d0, sacc0))
    return sacc


def _kth_smallest(d2, k):
    rr, ll = d2.shape
    lane = lax.broadcasted_iota(jnp.int32, (rr, ll), 1)

    def body(_, dw):
        m = jnp.min(dw, axis=1, keepdims=True)
        idx = jnp.min(jnp.where(dw == m, lane, ll), axis=1, keepdims=True)
        return jnp.where(lane == idx, jnp.float32(_BIG), dw)

    dw = lax.fori_loop(0, k - 1, body, d2)
    return jnp.min(dw, axis=1, keepdims=True)


def _dot(a, b):
    return jnp.dot(a, b, preferred_element_type=jnp.float32,
                   precision=lax.Precision.HIGHEST)


def _body(xc_ref, w1_ref, b1_ref, w2_ref, b2_ref, w3_ref, b3_ref,
          d1_ref, bd1_ref, h1_ref, bh1_ref, d2w_ref, bd2_ref,
          bh2k_ref, bh2t_ref, bd3k_ref, bd3t_ref, repd_ref, rep3_ref,
          out_ref):
    xs = [xc_ref[4 * c:4 * c + 4, :] for c in range(3)]

    sacc1 = _fps(xs[0], xs[1], xs[2], _NS1, _NS1P, None)
    st1 = _transpose(sacc1)

    d2b = []
    for b in range(_B):
        sc = [st1[:, 4 * c + b:4 * c + b + 1] for c in range(3)]
        xb = [xs[c][b:b + 1, :] for c in range(3)]
        d2b.append((sc[0] - xb[0]) ** 2 + (sc[1] - xb[1]) ** 2
                   + (sc[2] - xb[2]) ** 2)
    d2a = jnp.concatenate(d2b, axis=0)
    t1 = _kth_smallest(d2a, _K)
    sel1 = d2a <= jnp.minimum(t1, jnp.float32(_R1 * _R1))

    inv1 = jnp.float32(1.0 / _R1)
    srow1 = lax.broadcasted_iota(jnp.int32, (_NS1P, 1), 0)
    feat1 = []
    for b in range(_B):
        selb = sel1[_NS1P * b:_NS1P * (b + 1), :]
        sc = [st1[:, 4 * c + b:4 * c + b + 1] for c in range(3)]
        swb = (sc[0] * w1_ref[0:1, :] + sc[1] * w1_ref[1:2, :]
               + sc[2] * w1_ref[2:3, :])
        cols = []
        for f in range(5):
            xwf = (xs[0][b:b + 1, :] * w1_ref[0:1, f:f + 1]
                   + xs[1][b:b + 1, :] * w1_ref[1:2, f:f + 1]
                   + xs[2][b:b + 1, :] * w1_ref[2:3, f:f + 1])
            hf = jnp.maximum(xwf * inv1 - swb[:, f:f + 1] * inv1
                             + b1_ref[0:1, f:f + 1], 0.0)
            cols.append(jnp.max(jnp.where(selb, hf, jnp.float32(_NEG)),
                                axis=1, keepdims=True))
        fb = jnp.concatenate(cols, axis=1)
        feat1.append(jnp.where(srow1 < _NS1, fb, 0.0))

    lane2 = lax.broadcasted_iota(jnp.int32, (_B, _NS1P), 1)
    sacc2 = _fps(sacc1[0:4, :], sacc1[4:8, :], sacc1[8:12, :],
                 _NS2, _NS2P, lane2 < _NS1)
    st2 = _transpose(sacc2)

    d22b = []
    for b in range(_B):
        s2c = [st2[:, 4 * c + b:4 * c + b + 1] for c in range(3)]
        srw = [sacc1[4 * c + b:4 * c + b + 1, :] for c in range(3)]
        d22b.append((s2c[0] - srw[0]) ** 2 + (s2c[1] - srw[1]) ** 2
                    + (s2c[2] - srw[2]) ** 2)
    d22a = jnp.concatenate(d22b, axis=0)
    t2 = _kth_smallest(d22a, _K)
    sel2 = d22a <= jnp.minimum(t2, jnp.float32(_R2 * _R2))

    inv2 = jnp.float32(1.0 / _R2)
    srow2 = lax.broadcasted_iota(jnp.int32, (_NS2P, 1), 0)
    lats = []
    for b in range(_B):
        fb = feat1[b]
        sc = [st1[:, 4 * c + b:4 * c + b + 1] for c in range(3)]
        ab = fb[:, 0:1] * w2_ref[0:1, :]
        for k in range(1, 5):
            ab = ab + fb[:, k:k + 1] * w2_ref[k:k + 1, :]
        for c in range(3):
            ab = ab + (sc[c] * inv2) * w2_ref[5 + c:6 + c, :]
        abt = _transpose(ab)
        s2c = [st2[:, 4 * c + b:4 * c + b + 1] for c in range(3)]
        cb = ((s2c[0] * inv2) * w2_ref[5:6, :]
              + (s2c[1] * inv2) * w2_ref[6:7, :]
              + (s2c[2] * inv2) * w2_ref[7:8, :])
        selb = sel2[_NS2P * b:_NS2P * (b + 1), :]
        cols = []
        for f in range(25):
            hf = jnp.maximum(abt[f:f + 1, :] - cb[:, f:f + 1]
                             + b2_ref[0:1, f:f + 1], 0.0)
            cols.append(jnp.max(jnp.where(selb, hf, jnp.float32(_NEG)),
                                axis=1, keepdims=True))
        f2b = jnp.concatenate(cols, axis=1)
        s2mat = jnp.concatenate(s2c, axis=1)
        h3in = jnp.concatenate([f2b, s2mat * jnp.float32(1.0 / _R3)],
                               axis=1)
        h3 = jnp.maximum(_dot(h3in, w3_ref[:, :]) + b3_ref[0:1, :], 0.0)
        h3 = jnp.where(srow2 < _NS2, h3, jnp.float32(_NEG))
        lats.append(jnp.max(h3, axis=0, keepdims=True))
    latent = jnp.concatenate(lats, axis=0)

    g1w = jnp.maximum(_dot(latent, d1_ref[:, :]) + bd1_ref[0:1, :], 0.0)
    for d in range(_NBD):
        g1b = g1w[:, 25 * d:25 * (d + 1)]
        dec = _dot(g1b, h1_ref[:, :]) + bh1_ref[0:1, :]
        g2b = jnp.maximum(_dot(g1b, d2w_ref[:, :])
                          + bd2_ref[0:1, :], 0.0)
        dec2 = _dot(g2b, bh2k_ref[:, :]) + bh2t_ref[0:1, :]
        dec3 = _dot(g2b, bd3k_ref[:, :]) + bd3t_ref[0:1, :]
        so = (_dot(dec, repd_ref[:, :]) * jnp.float32(_R3) + dec2) \
            * jnp.float32(_R2)
        out_ref[:, 1200 * d:1200 * (d + 1)] = \
            (_dot(so, rep3_ref[:, :]) + dec3) * jnp.float32(_R1)


def kernel(points, batch, W1, b1, W2, b2, W3, b3, D1, bD1, H1, bH1,
           D2, bD2, H2, bH2, D3, bD3):
    del batch
    f32 = jnp.float32
    xc = points.reshape(_B, _N, 3).transpose(2, 0, 1).reshape(3 * _B, _N)
    eye20 = jnp.eye(_NB2, dtype=f32)
    bh2k = jnp.kron(eye20, H2.astype(f32))
    bd3k = jnp.kron(eye20, D3.astype(f32))
    repd = jnp.kron(jnp.ones((1, _NB2), f32), jnp.eye(3, dtype=f32))
    rep3 = jnp.kron(eye20, repd)
    bh2t = jnp.tile(bH2.reshape(1, 3), (1, _NB2))
    bd3t = jnp.tile(bD3.reshape(1, 60), (1, _NB2))

    out = pl.pallas_call(
        _body,
        out_shape=jax.ShapeDtypeStruct((_B, _NBD * _NB2 * _NB1 * 3), f32),
    )(xc, W1, b1.reshape(1, -1), W2, b2.reshape(1, -1),
      W3, b3.reshape(1, -1), D1, bD1.reshape(1, -1),
      H1, bH1.reshape(1, -1), D2, bD2.reshape(1, -1),
      bh2k, bh2t, bd3k, bd3t, repd, rep3)
    return out.reshape(_B * _NBD * _NB2 * _NB1, 3)


# speedup vs baseline: 1.4713x; 1.4713x over previous
import jax
import jax.numpy as jnp
from jax import lax
from jax.experimental import pallas as pl

_B = 4
_N = 2048
_NS1, _NS1P = 102, 104
_NS2, _NS2P = 5, 8
_K = 32
_R1, _R2, _R3 = 0.3, 1.0, 2.0
_NBD, _NB2, _NB1 = 5, 20, 20
_PAD = 1.0e4
_BIG = 1.0e30
_NEG = -1.0e30


def _transpose(a):
    c = a.shape[1]
    eye = (lax.broadcasted_iota(jnp.int32, (c, c), 0)
           == lax.broadcasted_iota(jnp.int32, (c, c), 1)).astype(jnp.float32)
    return lax.dot_general(eye, a, (((1,), (1,)), ((), ())),
                           preferred_element_type=jnp.float32,
                           precision=lax.Precision.HIGHEST)


def _fps(cx, cy, cz, nsamp, nslots, lane_valid):
    bb, ll = cx.shape
    lane = lax.broadcasted_iota(jnp.int32, (bb, ll), 1)
    slot = lax.broadcasted_iota(jnp.int32, (3 * bb, nslots), 1)
    p0x, p0y, p0z = cx[:, 0:1], cy[:, 0:1], cz[:, 0:1]
    d0 = (cx - p0x) ** 2 + (cy - p0y) ** 2 + (cz - p0z) ** 2
    if lane_valid is not None:
        d0 = jnp.where(lane_valid, d0, _NEG)
    sacc0 = jnp.where(slot == 0,
                      jnp.concatenate([p0x, p0y, p0z], axis=0),
                      jnp.float32(_PAD))

    def step(i, d, sacc, oh):
        px = jnp.sum(jnp.where(oh, cx, 0.0), axis=1, keepdims=True)
        py = jnp.sum(jnp.where(oh, cy, 0.0), axis=1, keepdims=True)
        pz = jnp.sum(jnp.where(oh, cz, 0.0), axis=1, keepdims=True)
        nd = (cx - px) ** 2 + (cy - py) ** 2 + (cz - pz) ** 2
        d = jnp.minimum(d, nd)
        sacc = jnp.where(slot == i,
                         jnp.concatenate([px, py, pz], axis=0), sacc)
        return d, sacc

    def body_fast(i, carry):
        d, sacc, bad = carry
        m = jnp.max(d, axis=1, keepdims=True)
        oh = d == m
        cnt = jnp.sum(jnp.where(oh, 1.0, 0.0), axis=1, keepdims=True)
        bad = jnp.maximum(bad, cnt)
        d, sacc = step(i, d, sacc, oh)
        return d, sacc, bad

    def body_exact(i, carry):
        d, sacc = carry
        m = jnp.max(d, axis=1, keepdims=True)
        idx = jnp.min(jnp.where(d == m, lane, ll), axis=1, keepdims=True)
        d, sacc = step(i, d, sacc, lane == idx)
        return d, sacc

    _, sacc, bad = lax.fori_loop(
        1, nsamp, body_fast, (d0, sacc0, jnp.zeros((bb, 1), jnp.float32)))
    return lax.cond(
        jnp.max(bad) <= 1.5,
        lambda: sacc,
        lambda: lax.fori_loop(1, nsamp, body_exact, (d0, sacc0))[1])


def _kth_exact(d2, k):
    rr, ll = d2.shape
    lane = lax.broadcasted_iota(jnp.int32, (rr, ll), 1)

    def body(_, dw):
        m = jnp.min(dw, axis=1, keepdims=True)
        idx = jnp.min(jnp.where(dw == m, lane, ll), axis=1, keepdims=True)
        return jnp.where(lane == idx, jnp.float32(_BIG), dw)

    dw = lax.fori_loop(0, k - 1, body, d2)
    return jnp.min(dw, axis=1, keepdims=True)


def _kth_smallest(d2, k, rsq):
    def body(_, dw):
        m = jnp.min(dw, axis=1, keepdims=True)
        return jnp.where(dw == m, jnp.float32(_BIG), dw)

    dw = lax.fori_loop(0, k - 1, body, d2)
    t = jnp.min(dw, axis=1, keepdims=True)
    cnt = jnp.sum(
        jnp.where(d2 <= jnp.minimum(t, jnp.float32(rsq)), 1.0, 0.0),
        axis=1, keepdims=True)
    return lax.cond(jnp.max(cnt) <= k + 0.5,
                    lambda: t,
                    lambda: _kth_exact(d2, k))


def _dot(a, b):
    return jnp.dot(a, b, preferred_element_type=jnp.float32,
                   precision=lax.Precision.HIGHEST)


def _body(xc_ref, w1_ref, b1_ref, w2_ref, b2_ref, w3_ref, b3_ref,
          d1_ref, bd1_ref, h1_ref, bh1_ref, d2w_ref, bd2_ref,
          bh2k_ref, bh2t_ref, bd3k_ref, bd3t_ref, repd_ref, rep3_ref,
          out_ref):
    xs = [xc_ref[4 * c:4 * c + 4, :] for c in range(3)]

    sacc1 = _fps(xs[0], xs[1], xs[2], _NS1, _NS1P, None)
    st1 = _transpose(sacc1)

    d2b = []
    for b in range(_B):
        sc = [st1[:, 4 * c + b:4 * c + b + 1] for c in range(3)]
        xb = [xs[c][b:b + 1, :] for c in range(3)]
        d2b.append((sc[0] - xb[0]) ** 2 + (sc[1] - xb[1]) ** 2
                   + (sc[2] - xb[2]) ** 2)
    d2a = jnp.concatenate(d2b, axis=0)
    t1 = _kth_smallest(d2a, _K, _R1 * _R1)
    sel1 = d2a <= jnp.minimum(t1, jnp.float32(_R1 * _R1))

    inv1 = jnp.float32(1.0 / _R1)
    srow1 = lax.broadcasted_iota(jnp.int32, (_NS1P, 1), 0)
    feat1 = []
    for b in range(_B):
        selb = sel1[_NS1P * b:_NS1P * (b + 1), :]
        sc = [st1[:, 4 * c + b:4 * c + b + 1] for c in range(3)]
        swb = (sc[0] * w1_ref[0:1, :] + sc[1] * w1_ref[1:2, :]
               + sc[2] * w1_ref[2:3, :])
        cols = []
        for f in range(5):
            xwf = (xs[0][b:b + 1, :] * w1_ref[0:1, f:f + 1]
                   + xs[1][b:b + 1, :] * w1_ref[1:2, f:f + 1]
                   + xs[2][b:b + 1, :] * w1_ref[2:3, f:f + 1])
            hf = jnp.maximum(xwf * inv1 - swb[:, f:f + 1] * inv1
                             + b1_ref[0:1, f:f + 1], 0.0)
            cols.append(jnp.max(jnp.where(selb, hf, jnp.float32(_NEG)),
                                axis=1, keepdims=True))
        fb = jnp.concatenate(cols, axis=1)
        feat1.append(jnp.where(srow1 < _NS1, fb, 0.0))

    lane2 = lax.broadcasted_iota(jnp.int32, (_B, _NS1P), 1)
    sacc2 = _fps(sacc1[0:4, :], sacc1[4:8, :], sacc1[8:12, :],
                 _NS2, _NS2P, lane2 < _NS1)
    st2 = _transpose(sacc2)

    d22b = []
    for b in range(_B):
        s2c = [st2[:, 4 * c + b:4 * c + b + 1] for c in range(3)]
        srw = [sacc1[4 * c + b:4 * c + b + 1, :] for c in range(3)]
        d22b.append((s2c[0] - srw[0]) ** 2 + (s2c[1] - srw[1]) ** 2
                    + (s2c[2] - srw[2]) ** 2)
    d22a = jnp.concatenate(d22b, axis=0)
    t2 = _kth_smallest(d22a, _K, _R2 * _R2)
    sel2 = d22a <= jnp.minimum(t2, jnp.float32(_R2 * _R2))

    inv2 = jnp.float32(1.0 / _R2)
    srow2 = lax.broadcasted_iota(jnp.int32, (_NS2P, 1), 0)
    lats = []
    for b in range(_B):
        fb = feat1[b]
        sc = [st1[:, 4 * c + b:4 * c + b + 1] for c in range(3)]
        ab = fb[:, 0:1] * w2_ref[0:1, :]
        for k in range(1, 5):
            ab = ab + fb[:, k:k + 1] * w2_ref[k:k + 1, :]
        for c in range(3):
            ab = ab + (sc[c] * inv2) * w2_ref[5 + c:6 + c, :]
        abt = _transpose(ab)
        s2c = [st2[:, 4 * c + b:4 * c + b + 1] for c in range(3)]
        cb = ((s2c[0] * inv2) * w2_ref[5:6, :]
              + (s2c[1] * inv2) * w2_ref[6:7, :]
              + (s2c[2] * inv2) * w2_ref[7:8, :])
        selb = sel2[_NS2P * b:_NS2P * (b + 1), :]
        cols = []
        for f in range(25):
            hf = jnp.maximum(abt[f:f + 1, :] - cb[:, f:f + 1]
                             + b2_ref[0:1, f:f + 1], 0.0)
            cols.append(jnp.max(jnp.where(selb, hf, jnp.float32(_NEG)),
                                axis=1, keepdims=True))
        f2b = jnp.concatenate(cols, axis=1)
        s2mat = jnp.concatenate(s2c, axis=1)
        h3in = jnp.concatenate([f2b, s2mat * jnp.float32(1.0 / _R3)],
                               axis=1)
        h3 = jnp.maximum(_dot(h3in, w3_ref[:, :]) + b3_ref[0:1, :], 0.0)
        h3 = jnp.where(srow2 < _NS2, h3, jnp.float32(_NEG))
        lats.append(jnp.max(h3, axis=0, keepdims=True))
    latent = jnp.concatenate(lats, axis=0)

    g1w = jnp.maximum(_dot(latent, d1_ref[:, :]) + bd1_ref[0:1, :], 0.0)
    for d in range(_NBD):
        g1b = g1w[:, 25 * d:25 * (d + 1)]
        dec = _dot(g1b, h1_ref[:, :]) + bh1_ref[0:1, :]
        g2b = jnp.maximum(_dot(g1b, d2w_ref[:, :])
                          + bd2_ref[0:1, :], 0.0)
        dec2 = _dot(g2b, bh2k_ref[:, :]) + bh2t_ref[0:1, :]
        dec3 = _dot(g2b, bd3k_ref[:, :]) + bd3t_ref[0:1, :]
        so = (_dot(dec, repd_ref[:, :]) * jnp.float32(_R3) + dec2) \
            * jnp.float32(_R2)
        out_ref[:, 1200 * d:1200 * (d + 1)] = \
            (_dot(so, rep3_ref[:, :]) + dec3) * jnp.float32(_R1)


def kernel(points, batch, W1, b1, W2, b2, W3, b3, D1, bD1, H1, bH1,
           D2, bD2, H2, bH2, D3, bD3):
    del batch
    f32 = jnp.float32
    xc = points.reshape(_B, _N, 3).transpose(2, 0, 1).reshape(3 * _B, _N)
    eye20 = jnp.eye(_NB2, dtype=f32)
    bh2k = jnp.kron(eye20, H2.astype(f32))
    bd3k = jnp.kron(eye20, D3.astype(f32))
    repd = jnp.kron(jnp.ones((1, _NB2), f32), jnp.eye(3, dtype=f32))
    rep3 = jnp.kron(eye20, repd)
    bh2t = jnp.tile(bH2.reshape(1, 3), (1, _NB2))
    bd3t = jnp.tile(bD3.reshape(1, 60), (1, _NB2))

    out = pl.pallas_call(
        _body,
        out_shape=jax.ShapeDtypeStruct((_B, _NBD * _NB2 * _NB1 * 3), f32),
    )(xc, W1, b1.reshape(1, -1), W2, b2.reshape(1, -1),
      W3, b3.reshape(1, -1), D1, bD1.reshape(1, -1),
      H1, bH1.reshape(1, -1), D2, bD2.reshape(1, -1),
      bh2k, bh2t, bd3k, bd3t, repd, rep3)
    return out.reshape(_B * _NBD * _NB2 * _NB1, 3)


# speedup vs baseline: 1.5698x; 1.0669x over previous
import jax
import jax.numpy as jnp
from jax import lax
from jax.experimental import pallas as pl

_B = 4
_N = 2048
_NS1, _NS1P = 102, 104
_NS2, _NS2P = 5, 8
_K = 32
_R1, _R2, _R3 = 0.3, 1.0, 2.0
_NBD, _NB2, _NB1 = 5, 20, 20
_PAD = 1.0e4
_BIG = 1.0e30
_NEG = -1.0e30


def _transpose(a):
    c = a.shape[1]
    eye = (lax.broadcasted_iota(jnp.int32, (c, c), 0)
           == lax.broadcasted_iota(jnp.int32, (c, c), 1)).astype(jnp.float32)
    return lax.dot_general(eye, a, (((1,), (1,)), ((), ())),
                           preferred_element_type=jnp.float32,
                           precision=lax.Precision.HIGHEST)


def _fps(cx, cy, cz, nsamp, nslots, lane_valid):
    bb, ll = cx.shape
    lane = lax.broadcasted_iota(jnp.int32, (bb, ll), 1)
    slot = lax.broadcasted_iota(jnp.int32, (3 * bb, nslots), 1)
    p0x, p0y, p0z = cx[:, 0:1], cy[:, 0:1], cz[:, 0:1]
    d0 = (cx - p0x) ** 2 + (cy - p0y) ** 2 + (cz - p0z) ** 2
    if lane_valid is not None:
        d0 = jnp.where(lane_valid, d0, _NEG)
    sacc0 = jnp.where(slot == 0,
                      jnp.concatenate([p0x, p0y, p0z], axis=0),
                      jnp.float32(_PAD))

    def step(i, d, sacc, oh):
        px = jnp.sum(jnp.where(oh, cx, 0.0), axis=1, keepdims=True)
        py = jnp.sum(jnp.where(oh, cy, 0.0), axis=1, keepdims=True)
        pz = jnp.sum(jnp.where(oh, cz, 0.0), axis=1, keepdims=True)
        nd = (cx - px) ** 2 + (cy - py) ** 2 + (cz - pz) ** 2
        d = jnp.minimum(d, nd)
        sacc = jnp.where(slot == i,
                         jnp.concatenate([px, py, pz], axis=0), sacc)
        return d, sacc

    def body_fast(i, carry):
        d, m, sacc, bad = carry
        oh = d == m
        cnt = jnp.sum(jnp.where(oh, 1.0, 0.0), axis=1, keepdims=True)
        bad = jnp.maximum(bad, cnt)
        d, sacc = step(i, d, sacc, oh)
        return d, jnp.max(d, axis=1, keepdims=True), sacc, bad

    def body_exact(i, carry):
        d, sacc = carry
        m = jnp.max(d, axis=1, keepdims=True)
        idx = jnp.min(jnp.where(d == m, lane, ll), axis=1, keepdims=True)
        d, sacc = step(i, d, sacc, lane == idx)
        return d, sacc

    m0 = jnp.max(d0, axis=1, keepdims=True)
    _, _, sacc, bad = lax.fori_loop(
        1, nsamp, body_fast,
        (d0, m0, sacc0, jnp.zeros((bb, 1), jnp.float32)))
    return lax.cond(
        jnp.max(bad) <= 1.5,
        lambda: sacc,
        lambda: lax.fori_loop(1, nsamp, body_exact, (d0, sacc0))[1])


def _kth_exact(d2, k):
    rr, ll = d2.shape
    lane = lax.broadcasted_iota(jnp.int32, (rr, ll), 1)

    def body(_, dw):
        m = jnp.min(dw, axis=1, keepdims=True)
        idx = jnp.min(jnp.where(dw == m, lane, ll), axis=1, keepdims=True)
        return jnp.where(lane == idx, jnp.float32(_BIG), dw)

    dw = lax.fori_loop(0, k - 1, body, d2)
    return jnp.min(dw, axis=1, keepdims=True)


def _kth_smallest(d2, k, rsq):
    def body(_, carry):
        dw, m = carry
        dwn = jnp.where(dw == m, jnp.float32(_BIG), dw)
        return dwn, jnp.min(dwn, axis=1, keepdims=True)

    m0 = jnp.min(d2, axis=1, keepdims=True)
    _, t = lax.fori_loop(0, k - 1, body, (d2, m0))
    cnt = jnp.sum(
        jnp.where(d2 <= jnp.minimum(t, jnp.float32(rsq)), 1.0, 0.0),
        axis=1, keepdims=True)
    return lax.cond(jnp.max(cnt) <= k + 0.5,
                    lambda: t,
                    lambda: _kth_exact(d2, k))


def _dot(a, b):
    return jnp.dot(a, b, preferred_element_type=jnp.float32,
                   precision=lax.Precision.HIGHEST)


def _body(xc_ref, w1_ref, b1_ref, w2_ref, b2_ref, w3_ref, b3_ref,
          d1_ref, bd1_ref, h1_ref, bh1_ref, d2w_ref, bd2_ref,
          bh2k_ref, bh2t_ref, bd3k_ref, bd3t_ref, repd_ref, rep3_ref,
          out_ref):
    xs = [xc_ref[4 * c:4 * c + 4, :] for c in range(3)]

    sacc1 = _fps(xs[0], xs[1], xs[2], _NS1, _NS1P, None)
    st1 = _transpose(sacc1)

    d2b = []
    for b in range(_B):
        sc = [st1[:, 4 * c + b:4 * c + b + 1] for c in range(3)]
        xb = [xs[c][b:b + 1, :] for c in range(3)]
        d2b.append((sc[0] - xb[0]) ** 2 + (sc[1] - xb[1]) ** 2
                   + (sc[2] - xb[2]) ** 2)
    d2a = jnp.concatenate(d2b, axis=0)
    t1 = _kth_smallest(d2a, _K, _R1 * _R1)
    sel1 = d2a <= jnp.minimum(t1, jnp.float32(_R1 * _R1))

    inv1 = jnp.float32(1.0 / _R1)
    srow1 = lax.broadcasted_iota(jnp.int32, (_NS1P, 1), 0)
    feat1 = []
    for b in range(_B):
        selb = sel1[_NS1P * b:_NS1P * (b + 1), :]
        sc = [st1[:, 4 * c + b:4 * c + b + 1] for c in range(3)]
        swb = (sc[0] * w1_ref[0:1, :] + sc[1] * w1_ref[1:2, :]
               + sc[2] * w1_ref[2:3, :])
        cols = []
        for f in range(5):
            xwf = (xs[0][b:b + 1, :] * w1_ref[0:1, f:f + 1]
                   + xs[1][b:b + 1, :] * w1_ref[1:2, f:f + 1]
                   + xs[2][b:b + 1, :] * w1_ref[2:3, f:f + 1])
            mm = jnp.max(jnp.where(selb, xwf, jnp.float32(_NEG)),
                         axis=1, keepdims=True)
            cols.append(jnp.maximum(mm * inv1 - swb[:, f:f + 1] * inv1
                                    + b1_ref[0:1, f:f + 1], 0.0))
        fb = jnp.concatenate(cols, axis=1)
        feat1.append(jnp.where(srow1 < _NS1, fb, 0.0))

    lane2 = lax.broadcasted_iota(jnp.int32, (_B, _NS1P), 1)
    sacc2 = _fps(sacc1[0:4, :], sacc1[4:8, :], sacc1[8:12, :],
                 _NS2, _NS2P, lane2 < _NS1)
    st2 = _transpose(sacc2)

    d22b = []
    for b in range(_B):
        s2c = [st2[:, 4 * c + b:4 * c + b + 1] for c in range(3)]
        srw = [sacc1[4 * c + b:4 * c + b + 1, :] for c in range(3)]
        d22b.append((s2c[0] - srw[0]) ** 2 + (s2c[1] - srw[1]) ** 2
                    + (s2c[2] - srw[2]) ** 2)
    d22a = jnp.concatenate(d22b, axis=0)
    t2 = _kth_smallest(d22a, _K, _R2 * _R2)
    sel2 = d22a <= jnp.minimum(t2, jnp.float32(_R2 * _R2))

    inv2 = jnp.float32(1.0 / _R2)
    srow2 = lax.broadcasted_iota(jnp.int32, (_NS2P, 1), 0)
    lats = []
    for b in range(_B):
        fb = feat1[b]
        sc = [st1[:, 4 * c + b:4 * c + b + 1] for c in range(3)]
        ab = fb[:, 0:1] * w2_ref[0:1, :]
        for k in range(1, 5):
            ab = ab + fb[:, k:k + 1] * w2_ref[k:k + 1, :]
        for c in range(3):
            ab = ab + (sc[c] * inv2) * w2_ref[5 + c:6 + c, :]
        abt = _transpose(ab)
        s2c = [st2[:, 4 * c + b:4 * c + b + 1] for c in range(3)]
        cb = ((s2c[0] * inv2) * w2_ref[5:6, :]
              + (s2c[1] * inv2) * w2_ref[6:7, :]
              + (s2c[2] * inv2) * w2_ref[7:8, :])
        selb = sel2[_NS2P * b:_NS2P * (b + 1), :]
        cols = []
        for f in range(25):
            mm = jnp.max(jnp.where(selb, abt[f:f + 1, :],
                                   jnp.float32(_NEG)),
                         axis=1, keepdims=True)
            cols.append(jnp.maximum(mm - cb[:, f:f + 1]
                                    + b2_ref[0:1, f:f + 1], 0.0))
        f2b = jnp.concatenate(cols, axis=1)
        s2mat = jnp.concatenate(s2c, axis=1)
        h3in = jnp.concatenate([f2b, s2mat * jnp.float32(1.0 / _R3)],
                               axis=1)
        h3 = jnp.maximum(_dot(h3in, w3_ref[:, :]) + b3_ref[0:1, :], 0.0)
        h3 = jnp.where(srow2 < _NS2, h3, jnp.float32(_NEG))
        lats.append(jnp.max(h3, axis=0, keepdims=True))
    latent = jnp.concatenate(lats, axis=0)

    g1w = jnp.maximum(_dot(latent, d1_ref[:, :]) + bd1_ref[0:1, :], 0.0)
    for d in range(_NBD):
        g1b = g1w[:, 25 * d:25 * (d + 1)]
        dec = _dot(g1b, h1_ref[:, :]) + bh1_ref[0:1, :]
        g2b = jnp.maximum(_dot(g1b, d2w_ref[:, :])
                          + bd2_ref[0:1, :], 0.0)
        dec2 = _dot(g2b, bh2k_ref[:, :]) + bh2t_ref[0:1, :]
        dec3 = _dot(g2b, bd3k_ref[:, :]) + bd3t_ref[0:1, :]
        so = (_dot(dec, repd_ref[:, :]) * jnp.float32(_R3) + dec2) \
            * jnp.float32(_R2)
        out_ref[:, 1200 * d:1200 * (d + 1)] = \
            (_dot(so, rep3_ref[:, :]) + dec3) * jnp.float32(_R1)


def kernel(points, batch, W1, b1, W2, b2, W3, b3, D1, bD1, H1, bH1,
           D2, bD2, H2, bH2, D3, bD3):
    del batch
    f32 = jnp.float32
    xc = points.reshape(_B, _N, 3).transpose(2, 0, 1).reshape(3 * _B, _N)
    eye20 = jnp.eye(_NB2, dtype=f32)
    bh2k = jnp.kron(eye20, H2.astype(f32))
    bd3k = jnp.kron(eye20, D3.astype(f32))
    repd = jnp.kron(jnp.ones((1, _NB2), f32), jnp.eye(3, dtype=f32))
    rep3 = jnp.kron(eye20, repd)
    bh2t = jnp.tile(bH2.reshape(1, 3), (1, _NB2))
    bd3t = jnp.tile(bD3.reshape(1, 60), (1, _NB2))

    out = pl.pallas_call(
        _body,
        out_shape=jax.ShapeDtypeStruct((_B, _NBD * _NB2 * _NB1 * 3), f32),
    )(xc, W1, b1.reshape(1, -1), W2, b2.reshape(1, -1),
      W3, b3.reshape(1, -1), D1, bD1.reshape(1, -1),
      H1, bH1.reshape(1, -1), D2, bD2.reshape(1, -1),
      bh2k, bh2t, bd3k, bd3t, repd, rep3)
    return out.reshape(_B * _NBD * _NB2 * _NB1, 3)


# speedup vs baseline: 1.5873x; 1.0112x over previous
import jax
import jax.numpy as jnp
from jax import lax
from jax.experimental import pallas as pl

_B = 4
_N = 2048
_NS1, _NS1P = 102, 104
_NS2, _NS2P = 5, 8
_K = 32
_R1, _R2, _R3 = 0.3, 1.0, 2.0
_NBD, _NB2, _NB1 = 5, 20, 20
_PAD = 1.0e4
_BIG = 1.0e30
_NEG = -1.0e30


def _transpose(a):
    c = a.shape[1]
    eye = (lax.broadcasted_iota(jnp.int32, (c, c), 0)
           == lax.broadcasted_iota(jnp.int32, (c, c), 1)).astype(jnp.float32)
    return lax.dot_general(eye, a, (((1,), (1,)), ((), ())),
                           preferred_element_type=jnp.float32,
                           precision=lax.Precision.HIGHEST)


def _fps(cx, cy, cz, nsamp, nslots, lane_valid):
    bb, ll = cx.shape
    lane = lax.broadcasted_iota(jnp.int32, (bb, ll), 1)
    slot = lax.broadcasted_iota(jnp.int32, (3 * bb, nslots), 1)
    p0x, p0y, p0z = cx[:, 0:1], cy[:, 0:1], cz[:, 0:1]
    d0 = (cx - p0x) ** 2 + (cy - p0y) ** 2 + (cz - p0z) ** 2
    if lane_valid is not None:
        d0 = jnp.where(lane_valid, d0, _NEG)
    sacc0 = jnp.where(slot == 0,
                      jnp.concatenate([p0x, p0y, p0z], axis=0),
                      jnp.float32(_PAD))

    def step(i, d, sacc, oh):
        px = jnp.sum(jnp.where(oh, cx, 0.0), axis=1, keepdims=True)
        py = jnp.sum(jnp.where(oh, cy, 0.0), axis=1, keepdims=True)
        pz = jnp.sum(jnp.where(oh, cz, 0.0), axis=1, keepdims=True)
        nd = (cx - px) ** 2 + (cy - py) ** 2 + (cz - pz) ** 2
        d = jnp.minimum(d, nd)
        sacc = jnp.where(slot == i,
                         jnp.concatenate([px, py, pz], axis=0), sacc)
        return d, sacc

    def body_fast(i, carry):
        d, m, sacc, bad = carry
        oh = d == m
        cnt = jnp.sum(jnp.where(oh, 1.0, 0.0), axis=1, keepdims=True)
        bad = jnp.maximum(bad, cnt)
        d, sacc = step(i, d, sacc, oh)
        return d, jnp.max(d, axis=1, keepdims=True), sacc, bad

    def body_exact(i, carry):
        d, sacc = carry
        m = jnp.max(d, axis=1, keepdims=True)
        idx = jnp.min(jnp.where(d == m, lane, ll), axis=1, keepdims=True)
        d, sacc = step(i, d, sacc, lane == idx)
        return d, sacc

    m0 = jnp.max(d0, axis=1, keepdims=True)
    _, _, sacc, bad = lax.fori_loop(
        1, nsamp, body_fast,
        (d0, m0, sacc0, jnp.zeros((bb, 1), jnp.float32)))
    return lax.cond(
        jnp.max(bad) <= 1.5,
        lambda: sacc,
        lambda: lax.fori_loop(1, nsamp, body_exact, (d0, sacc0))[1])


def _kth_exact(d2, k):
    rr, ll = d2.shape
    lane = lax.broadcasted_iota(jnp.int32, (rr, ll), 1)

    def body(_, dw):
        m = jnp.min(dw, axis=1, keepdims=True)
        idx = jnp.min(jnp.where(dw == m, lane, ll), axis=1, keepdims=True)
        return jnp.where(lane == idx, jnp.float32(_BIG), dw)

    dw = lax.fori_loop(0, k - 1, body, d2)
    return jnp.min(dw, axis=1, keepdims=True)


def _kth_smallest(d2, k, rsq):
    def body(_, carry):
        dw, m = carry
        dwn = jnp.where(dw == m, jnp.float32(_BIG), dw)
        return dwn, jnp.min(dwn, axis=1, keepdims=True)

    m0 = jnp.min(d2, axis=1, keepdims=True)
    _, t = lax.fori_loop(0, k - 1, body, (d2, m0))
    cnt = jnp.sum(
        jnp.where(d2 <= jnp.minimum(t, jnp.float32(rsq)), 1.0, 0.0),
        axis=1, keepdims=True)
    return lax.cond(jnp.max(cnt) <= k + 0.5,
                    lambda: t,
                    lambda: _kth_exact(d2, k))


def _dot(a, b):
    return jnp.dot(a, b, preferred_element_type=jnp.float32,
                   precision=lax.Precision.HIGHEST)


def _body(xc_ref, w1_ref, b1_ref, w2_ref, b2_ref, w3_ref, b3_ref,
          d1_ref, bd1_ref, h1_ref, bh1_ref, d2w_ref, bd2_ref,
          bh2k_ref, bh2t_ref, bd3k_ref, bd3t_ref, repd_ref, rep3_ref,
          repb_ref, out_ref):
    xs = [xc_ref[4 * c:4 * c + 4, :] for c in range(3)]

    sacc1 = _fps(xs[0], xs[1], xs[2], _NS1, _NS1P, None)
    st1 = _transpose(sacc1)

    d2b = []
    for b in range(_B):
        sc = [st1[:, 4 * c + b:4 * c + b + 1] for c in range(3)]
        xb = [xs[c][b:b + 1, :] for c in range(3)]
        d2b.append((sc[0] - xb[0]) ** 2 + (sc[1] - xb[1]) ** 2
                   + (sc[2] - xb[2]) ** 2)
    d2a = jnp.concatenate(d2b, axis=0)
    t1 = _kth_smallest(d2a, _K, _R1 * _R1)
    sel1 = d2a <= jnp.minimum(t1, jnp.float32(_R1 * _R1))

    inv1 = jnp.float32(1.0 / _R1)
    srow1 = lax.broadcasted_iota(jnp.int32, (_NS1P, 1), 0)
    feat1 = []
    for b in range(_B):
        selb = sel1[_NS1P * b:_NS1P * (b + 1), :]
        sc = [st1[:, 4 * c + b:4 * c + b + 1] for c in range(3)]
        swb = (sc[0] * w1_ref[0:1, :] + sc[1] * w1_ref[1:2, :]
               + sc[2] * w1_ref[2:3, :])
        cols = []
        for f in range(5):
            xwf = (xs[0][b:b + 1, :] * w1_ref[0:1, f:f + 1]
                   + xs[1][b:b + 1, :] * w1_ref[1:2, f:f + 1]
                   + xs[2][b:b + 1, :] * w1_ref[2:3, f:f + 1])
            mm = jnp.max(jnp.where(selb, xwf, jnp.float32(_NEG)),
                         axis=1, keepdims=True)
            cols.append(jnp.maximum(mm * inv1 - swb[:, f:f + 1] * inv1
                                    + b1_ref[0:1, f:f + 1], 0.0))
        fb = jnp.concatenate(cols, axis=1)
        feat1.append(jnp.where(srow1 < _NS1, fb, 0.0))

    lane2 = lax.broadcasted_iota(jnp.int32, (_B, _NS1P), 1)
    sacc2 = _fps(sacc1[0:4, :], sacc1[4:8, :], sacc1[8:12, :],
                 _NS2, _NS2P, lane2 < _NS1)
    st2 = _transpose(sacc2)

    d22b = []
    for b in range(_B):
        s2c = [st2[:, 4 * c + b:4 * c + b + 1] for c in range(3)]
        srw = [sacc1[4 * c + b:4 * c + b + 1, :] for c in range(3)]
        d22b.append((s2c[0] - srw[0]) ** 2 + (s2c[1] - srw[1]) ** 2
                    + (s2c[2] - srw[2]) ** 2)
    d22a = jnp.concatenate(d22b, axis=0)
    t2 = _kth_smallest(d22a, _K, _R2 * _R2)
    sel2 = d22a <= jnp.minimum(t2, jnp.float32(_R2 * _R2))

    inv2 = jnp.float32(1.0 / _R2)
    srow2 = lax.broadcasted_iota(jnp.int32, (_NS2P, 1), 0)
    lats = []
    for b in range(_B):
        fb = feat1[b]
        sc = [st1[:, 4 * c + b:4 * c + b + 1] for c in range(3)]
        ab = fb[:, 0:1] * w2_ref[0:1, :]
        for k in range(1, 5):
            ab = ab + fb[:, k:k + 1] * w2_ref[k:k + 1, :]
        for c in range(3):
            ab = ab + (sc[c] * inv2) * w2_ref[5 + c:6 + c, :]
        abt = _transpose(ab)
        s2c = [st2[:, 4 * c + b:4 * c + b + 1] for c in range(3)]
        cb = ((s2c[0] * inv2) * w2_ref[5:6, :]
              + (s2c[1] * inv2) * w2_ref[6:7, :]
              + (s2c[2] * inv2) * w2_ref[7:8, :])
        selb = sel2[_NS2P * b:_NS2P * (b + 1), :]
        cols = []
        for f in range(25):
            mm = jnp.max(jnp.where(selb, abt[f:f + 1, :],
                                   jnp.float32(_NEG)),
                         axis=1, keepdims=True)
            cols.append(jnp.maximum(mm - cb[:, f:f + 1]
                                    + b2_ref[0:1, f:f + 1], 0.0))
        f2b = jnp.concatenate(cols, axis=1)
        s2mat = jnp.concatenate(s2c, axis=1)
        h3in = jnp.concatenate([f2b, s2mat * jnp.float32(1.0 / _R3)],
                               axis=1)
        h3 = jnp.maximum(_dot(h3in, w3_ref[:, :]) + b3_ref[0:1, :], 0.0)
        h3 = jnp.where(srow2 < _NS2, h3, jnp.float32(_NEG))
        lats.append(jnp.max(h3, axis=0, keepdims=True))
    latent = jnp.concatenate(lats, axis=0)

    lat_rep = _dot(repb_ref[:, :], latent)
    g1w = jnp.maximum(_dot(lat_rep, d1_ref[:, :])
                      + bd1_ref[0:1, :], 0.0)
    rowd = lax.broadcasted_iota(jnp.int32, (_B * _NBD, 1), 0) % _NBD
    g1 = jnp.where(rowd == 0, g1w[:, 0:25], 0.0)
    for d in range(1, _NBD):
        g1 = jnp.where(rowd == d, g1w[:, 25 * d:25 * (d + 1)], g1)
    dec = _dot(g1, h1_ref[:, :]) + bh1_ref[0:1, :]
    g2 = jnp.maximum(_dot(g1, d2w_ref[:, :])
                     + bd2_ref[0:1, :], 0.0)
    dec2 = _dot(g2, bh2k_ref[:, :]) + bh2t_ref[0:1, :]
    dec3 = _dot(g2, bd3k_ref[:, :]) + bd3t_ref[0:1, :]
    so = (_dot(dec, repd_ref[:, :]) * jnp.float32(_R3) + dec2) \
        * jnp.float32(_R2)
    out_ref[:, :] = (_dot(so, rep3_ref[:, :]) + dec3) * jnp.float32(_R1)


def kernel(points, batch, W1, b1, W2, b2, W3, b3, D1, bD1, H1, bH1,
           D2, bD2, H2, bH2, D3, bD3):
    del batch
    f32 = jnp.float32
    xc = points.reshape(_B, _N, 3).transpose(2, 0, 1).reshape(3 * _B, _N)
    eye20 = jnp.eye(_NB2, dtype=f32)
    bh2k = jnp.kron(eye20, H2.astype(f32))
    bd3k = jnp.kron(eye20, D3.astype(f32))
    repd = jnp.kron(jnp.ones((1, _NB2), f32), jnp.eye(3, dtype=f32))
    rep3 = jnp.kron(eye20, repd)
    bh2t = jnp.tile(bH2.reshape(1, 3), (1, _NB2))
    bd3t = jnp.tile(bD3.reshape(1, 60), (1, _NB2))
    repb = jnp.kron(jnp.eye(_B, dtype=f32), jnp.ones((_NBD, 1), f32))

    out = pl.pallas_call(
        _body,
        out_shape=jax.ShapeDtypeStruct((_B * _NBD, _NB2 * _NB1 * 3), f32),
    )(xc, W1, b1.reshape(1, -1), W2, b2.reshape(1, -1),
      W3, b3.reshape(1, -1), D1, bD1.reshape(1, -1),
      H1, bH1.reshape(1, -1), D2, bD2.reshape(1, -1),
      bh2k, bh2t, bd3k, bd3t, repd, rep3, repb)
    return out.reshape(_B * _NBD * _NB2 * _NB1, 3)


# speedup vs baseline: 1.5877x; 1.0003x over previous
import jax
import jax.numpy as jnp
from jax import lax
from jax.experimental import pallas as pl

_B = 4
_N = 2048
_NS1, _NS1P = 102, 104
_NS2, _NS2P = 5, 8
_K = 32
_R1, _R2, _R3 = 0.3, 1.0, 2.0
_NBD, _NB2, _NB1 = 5, 20, 20
_PAD = 1.0e4
_BIG = 1.0e30
_NEG = -1.0e30


def _transpose(a):
    c = a.shape[1]
    eye = (lax.broadcasted_iota(jnp.int32, (c, c), 0)
           == lax.broadcasted_iota(jnp.int32, (c, c), 1)).astype(jnp.float32)
    return lax.dot_general(eye, a, (((1,), (1,)), ((), ())),
                           preferred_element_type=jnp.float32,
                           precision=lax.Precision.HIGHEST)


def _fps(cx, cy, cz, nsamp, nslots, lane_valid):
    bb, ll = cx.shape
    lane = lax.broadcasted_iota(jnp.int32, (bb, ll), 1)
    slot = lax.broadcasted_iota(jnp.int32, (3 * bb, nslots), 1)
    p0x, p0y, p0z = cx[:, 0:1], cy[:, 0:1], cz[:, 0:1]
    d0 = (cx - p0x) ** 2 + (cy - p0y) ** 2 + (cz - p0z) ** 2
    if lane_valid is not None:
        d0 = jnp.where(lane_valid, d0, _NEG)
    sacc0 = jnp.where(slot == 0,
                      jnp.concatenate([p0x, p0y, p0z], axis=0),
                      jnp.float32(_PAD))

    def step(i, d, sacc, oh):
        px = jnp.sum(jnp.where(oh, cx, 0.0), axis=1, keepdims=True)
        py = jnp.sum(jnp.where(oh, cy, 0.0), axis=1, keepdims=True)
        pz = jnp.sum(jnp.where(oh, cz, 0.0), axis=1, keepdims=True)
        nd = (cx - px) ** 2 + (cy - py) ** 2 + (cz - pz) ** 2
        d = jnp.minimum(d, nd)
        sacc = jnp.where(slot == i,
                         jnp.concatenate([px, py, pz], axis=0), sacc)
        return d, sacc

    def body_fast(i, carry):
        d, m, sacc, bad = carry
        oh = d == m
        cnt = jnp.sum(jnp.where(oh, 1.0, 0.0), axis=1, keepdims=True)
        bad = jnp.maximum(bad, cnt)
        d, sacc = step(i, d, sacc, oh)
        return d, jnp.max(d, axis=1, keepdims=True), sacc, bad

    def body_exact(i, carry):
        d, sacc = carry
        m = jnp.max(d, axis=1, keepdims=True)
        idx = jnp.min(jnp.where(d == m, lane, ll), axis=1, keepdims=True)
        d, sacc = step(i, d, sacc, lane == idx)
        return d, sacc

    m0 = jnp.max(d0, axis=1, keepdims=True)
    _, _, sacc, bad = lax.fori_loop(
        1, nsamp, body_fast,
        (d0, m0, sacc0, jnp.zeros((bb, 1), jnp.float32)))
    return lax.cond(
        jnp.max(bad) <= 1.5,
        lambda: sacc,
        lambda: lax.fori_loop(1, nsamp, body_exact, (d0, sacc0))[1])


def _kth_exact(d2, k):
    rr, ll = d2.shape
    lane = lax.broadcasted_iota(jnp.int32, (rr, ll), 1)

    def body(_, dw):
        m = jnp.min(dw, axis=1, keepdims=True)
        idx = jnp.min(jnp.where(dw == m, lane, ll), axis=1, keepdims=True)
        return jnp.where(lane == idx, jnp.float32(_BIG), dw)

    dw = lax.fori_loop(0, k - 1, body, d2)
    return jnp.min(dw, axis=1, keepdims=True)


def _topk_select(d2, k, rsq):
    rcap = jnp.float32(rsq)

    def body(_, carry):
        dw, m = carry
        dwn = jnp.where(dw == m, jnp.float32(_BIG), dw)
        return dwn, jnp.min(dwn, axis=1, keepdims=True)

    m0 = jnp.min(d2, axis=1, keepdims=True)
    _, t = lax.fori_loop(0, k - 1, body, (d2, m0))
    cnt = jnp.sum(
        jnp.where(d2 <= jnp.minimum(t, rcap), 1.0, 0.0),
        axis=1, keepdims=True)
    t = lax.cond(jnp.max(cnt) <= k + 0.5,
                 lambda: t,
                 lambda: _kth_exact(d2, k))
    return d2 <= jnp.minimum(t, rcap)


def _dot(a, b):
    return jnp.dot(a, b, preferred_element_type=jnp.float32,
                   precision=lax.Precision.HIGHEST)


def _body(xc_ref, w1_ref, b1_ref, w2_ref, b2_ref, w3_ref, b3_ref,
          d1_ref, bd1_ref, h1_ref, bh1_ref, d2w_ref, bd2_ref,
          bh2k_ref, bh2t_ref, bd3k_ref, bd3t_ref, repd_ref, rep3_ref,
          repb_ref, out_ref):
    xs = [xc_ref[4 * c:4 * c + 4, :] for c in range(3)]

    sacc1 = _fps(xs[0], xs[1], xs[2], _NS1, _NS1P, None)
    st1 = _transpose(sacc1)

    d2b = []
    for b in range(_B):
        sc = [st1[:, 4 * c + b:4 * c + b + 1] for c in range(3)]
        xb = [xs[c][b:b + 1, :] for c in range(3)]
        d2b.append((sc[0] - xb[0]) ** 2 + (sc[1] - xb[1]) ** 2
                   + (sc[2] - xb[2]) ** 2)
    d2a = jnp.concatenate(d2b, axis=0)
    sel1 = _topk_select(d2a, _K, _R1 * _R1)

    inv1 = jnp.float32(1.0 / _R1)
    srow1 = lax.broadcasted_iota(jnp.int32, (_NS1P, 1), 0)
    feat1 = []
    for b in range(_B):
        selb = sel1[_NS1P * b:_NS1P * (b + 1), :]
        sc = [st1[:, 4 * c + b:4 * c + b + 1] for c in range(3)]
        swb = (sc[0] * w1_ref[0:1, :] + sc[1] * w1_ref[1:2, :]
               + sc[2] * w1_ref[2:3, :])
        cols = []
        for f in range(5):
            xwf = (xs[0][b:b + 1, :] * w1_ref[0:1, f:f + 1]
                   + xs[1][b:b + 1, :] * w1_ref[1:2, f:f + 1]
                   + xs[2][b:b + 1, :] * w1_ref[2:3, f:f + 1])
            mm = jnp.max(jnp.where(selb, xwf, jnp.float32(_NEG)),
                         axis=1, keepdims=True)
            cols.append(jnp.maximum(mm * inv1 - swb[:, f:f + 1] * inv1
                                    + b1_ref[0:1, f:f + 1], 0.0))
        fb = jnp.concatenate(cols, axis=1)
        feat1.append(jnp.where(srow1 < _NS1, fb, 0.0))

    lane2 = lax.broadcasted_iota(jnp.int32, (_B, _NS1P), 1)
    sacc2 = _fps(sacc1[0:4, :], sacc1[4:8, :], sacc1[8:12, :],
                 _NS2, _NS2P, lane2 < _NS1)
    st2 = _transpose(sacc2)

    d22b = []
    for b in range(_B):
        s2c = [st2[:, 4 * c + b:4 * c + b + 1] for c in range(3)]
        srw = [sacc1[4 * c + b:4 * c + b + 1, :] for c in range(3)]
        d22b.append((s2c[0] - srw[0]) ** 2 + (s2c[1] - srw[1]) ** 2
                    + (s2c[2] - srw[2]) ** 2)
    d22a = jnp.concatenate(d22b, axis=0)
    sel2 = _topk_select(d22a, _K, _R2 * _R2)

    inv2 = jnp.float32(1.0 / _R2)
    srow2 = lax.broadcasted_iota(jnp.int32, (_NS2P, 1), 0)
    lats = []
    for b in range(_B):
        fb = feat1[b]
        sc = [st1[:, 4 * c + b:4 * c + b + 1] for c in range(3)]
        ab = fb[:, 0:1] * w2_ref[0:1, :]
        for k in range(1, 5):
            ab = ab + fb[:, k:k + 1] * w2_ref[k:k + 1, :]
        for c in range(3):
            ab = ab + (sc[c] * inv2) * w2_ref[5 + c:6 + c, :]
        abt = _transpose(ab)
        s2c = [st2[:, 4 * c + b:4 * c + b + 1] for c in range(3)]
        cb = ((s2c[0] * inv2) * w2_ref[5:6, :]
              + (s2c[1] * inv2) * w2_ref[6:7, :]
              + (s2c[2] * inv2) * w2_ref[7:8, :])
        selb = sel2[_NS2P * b:_NS2P * (b + 1), :]
        cols = []
        for f in range(25):
            mm = jnp.max(jnp.where(selb, abt[f:f + 1, :],
                                   jnp.float32(_NEG)),
                         axis=1, keepdims=True)
            cols.append(jnp.maximum(mm - cb[:, f:f + 1]
                                    + b2_ref[0:1, f:f + 1], 0.0))
        f2b = jnp.concatenate(cols, axis=1)
        s2mat = jnp.concatenate(s2c, axis=1)
        h3in = jnp.concatenate([f2b, s2mat * jnp.float32(1.0 / _R3)],
                               axis=1)
        h3 = jnp.maximum(_dot(h3in, w3_ref[:, :]) + b3_ref[0:1, :], 0.0)
        h3 = jnp.where(srow2 < _NS2, h3, jnp.float32(_NEG))
        lats.append(jnp.max(h3, axis=0, keepdims=True))
    latent = jnp.concatenate(lats, axis=0)

    lat_rep = _dot(repb_ref[:, :], latent)
    g1w = jnp.maximum(_dot(lat_rep, d1_ref[:, :])
                      + bd1_ref[0:1, :], 0.0)
    rowd = lax.broadcasted_iota(jnp.int32, (_B * _NBD, 1), 0) % _NBD
    g1 = jnp.where(rowd == 0, g1w[:, 0:25], 0.0)
    for d in range(1, _NBD):
        g1 = jnp.where(rowd == d, g1w[:, 25 * d:25 * (d + 1)], g1)
    dec = _dot(g1, h1_ref[:, :]) + bh1_ref[0:1, :]
    g2 = jnp.maximum(_dot(g1, d2w_ref[:, :])
                     + bd2_ref[0:1, :], 0.0)
    dec2 = _dot(g2, bh2k_ref[:, :]) + bh2t_ref[0:1, :]
    dec3 = _dot(g2, bd3k_ref[:, :]) + bd3t_ref[0:1, :]
    so = (_dot(dec, repd_ref[:, :]) * jnp.float32(_R3) + dec2) \
        * jnp.float32(_R2)
    out_ref[:, :] = (_dot(so, rep3_ref[:, :]) + dec3) * jnp.float32(_R1)


def kernel(points, batch, W1, b1, W2, b2, W3, b3, D1, bD1, H1, bH1,
           D2, bD2, H2, bH2, D3, bD3):
    del batch
    f32 = jnp.float32
    xc = points.reshape(_B, _N, 3).transpose(2, 0, 1).reshape(3 * _B, _N)
    eye20 = jnp.eye(_NB2, dtype=f32)
    bh2k = jnp.kron(eye20, H2.astype(f32))
    bd3k = jnp.kron(eye20, D3.astype(f32))
    repd = jnp.kron(jnp.ones((1, _NB2), f32), jnp.eye(3, dtype=f32))
    rep3 = jnp.kron(eye20, repd)
    bh2t = jnp.tile(bH2.reshape(1, 3), (1, _NB2))
    bd3t = jnp.tile(bD3.reshape(1, 60), (1, _NB2))
    repb = jnp.kron(jnp.eye(_B, dtype=f32), jnp.ones((_NBD, 1), f32))

    out = pl.pallas_call(
        _body,
        out_shape=jax.ShapeDtypeStruct((_B * _NBD, _NB2 * _NB1 * 3), f32),
    )(xc, W1, b1.reshape(1, -1), W2, b2.reshape(1, -1),
      W3, b3.reshape(1, -1), D1, bD1.reshape(1, -1),
      H1, bH1.reshape(1, -1), D2, bD2.reshape(1, -1),
      bh2k, bh2t, bd3k, bd3t, repd, rep3, repb)
    return out.reshape(_B * _NBD * _NB2 * _NB1, 3)
